# Initial kernel scaffold; baseline (speedup 1.0000x reference)
import jax
import jax.numpy as jnp
from jax.experimental import pallas as pl


def kernel(X, char_table, W_proj):
    raise NotImplementedError("write your pallas kernel here")


# speedup vs baseline: 2.8715x; 2.8715x over previous
import functools

import jax
import jax.numpy as jnp
from jax.experimental import pallas as pl
from jax.experimental.pallas import tpu as pltpu

B, L, W_CHARS = 1024, 50, 16
CHAR_SIZE = 128
CHAR_DIM = 30
HIDDEN = 1024
N_TOK = B * L

TB = 512


def _body(x_ref, ct_ref, wt_ref, o_ref):
    ct = ct_ref[:].astype(jnp.bfloat16)
    iot = jax.lax.broadcasted_iota(jnp.int32, (TB, CHAR_SIZE), 1)
    acc = jnp.zeros((TB, HIDDEN), jnp.float32)
    for w in range(W_CHARS):
        col = x_ref[:, w : w + 1]
        oh = (col == iot).astype(jnp.bfloat16)
        emb = jnp.dot(oh, ct, preferred_element_type=jnp.float32)
        acc += jnp.dot(
            emb.astype(jnp.bfloat16),
            wt_ref[w].astype(jnp.bfloat16),
            preferred_element_type=jnp.float32,
        )
    o_ref[:] = acc


@jax.jit
def kernel(X, char_table, W_proj):
    Xf = X.reshape(N_TOK, W_CHARS)
    Wt = W_proj.reshape(HIDDEN, W_CHARS, CHAR_DIM).transpose(1, 2, 0)
    grid = (N_TOK // TB,)
    out = pl.pallas_call(
        _body,
        grid=grid,
        in_specs=[
            pl.BlockSpec((TB, W_CHARS), lambda i: (i, 0)),
            pl.BlockSpec((CHAR_SIZE, CHAR_DIM), lambda i: (0, 0)),
            pl.BlockSpec((W_CHARS, CHAR_DIM, HIDDEN), lambda i: (0, 0, 0)),
        ],
        out_specs=pl.BlockSpec((TB, HIDDEN), lambda i: (i, 0)),
        out_shape=jax.ShapeDtypeStruct((N_TOK, HIDDEN), jnp.float32),
    )(Xf, char_table, Wt)
    return out.reshape(B, L, HIDDEN)


# speedup vs baseline: 2.9833x; 1.0389x over previous
import functools

import jax
import jax.numpy as jnp
from jax import lax
from jax.experimental import pallas as pl
from jax.experimental.pallas import tpu as pltpu
from jax.experimental.pallas import tpu_sc as plsc

B, L, W_CHARS = 1024, 50, 16
CHAR_SIZE = 128
CHAR_DIM = 30
CD_PAD = 32
HIDDEN = 1024
N_TOK = B * L
N_LOOK = N_TOK * W_CHARS

_NC, _NS = 2, 16
_NW = _NC * _NS
_IDX_W = 128
_RPW = N_LOOK // _NW // _IDX_W
_KF = 8
_NG = _RPW // _KF

_sc_mesh = plsc.VectorSubcoreMesh(
    core_axis_name="c", subcore_axis_name="s", num_cores=_NC, num_subcores=_NS
)


@functools.partial(
    pl.kernel,
    out_type=jax.ShapeDtypeStruct((N_LOOK, CD_PAD), jnp.float32),
    mesh=_sc_mesh,
    scratch_types=[
        pltpu.VMEM((_RPW, _IDX_W), jnp.int32),
        pltpu.VMEM((_KF * _IDX_W, CD_PAD), jnp.float32),
        pltpu.SemaphoreType.DMA,
    ],
    compiler_params=pltpu.CompilerParams(use_tc_tiling_on_sc=False),
)
def _sc_gather(idx_hbm, tab_hbm, emb_hbm, idx_v, rows_v, sem):
    wid = lax.axis_index("s") * _NC + lax.axis_index("c")
    pltpu.sync_copy(idx_hbm.at[wid], idx_v)
    row_base = wid * (_RPW * _IDX_W)

    @pl.loop(0, _NG)
    def _group(g):
        cps = [
            pltpu.async_copy(
                tab_hbm.at[idx_v.at[g * _KF + b]],
                rows_v.at[pl.ds(b * _IDX_W, _IDX_W)],
                sem,
            )
            for b in range(_KF)
        ]
        for c in cps:
            c.wait()
        pltpu.sync_copy(
            rows_v,
            emb_hbm.at[pl.ds(row_base + g * (_KF * _IDX_W), _KF * _IDX_W)],
        )


_TB = 512


def _mm_body(e_ref, wt_ref, o_ref):
    o_ref[:] = jnp.dot(
        e_ref[:].astype(jnp.bfloat16), wt_ref[:], preferred_element_type=jnp.float32
    )


@jax.jit
def kernel(X, char_table, W_proj):
    idx = X.reshape(_NW, _RPW, _IDX_W)
    tab32 = jnp.pad(char_table, ((0, 0), (0, CD_PAD - CHAR_DIM)))
    emb = _sc_gather(idx, tab32)

    wt = jnp.pad(
        W_proj.reshape(HIDDEN, W_CHARS, CHAR_DIM),
        ((0, 0), (0, 0), (0, CD_PAD - CHAR_DIM)),
    ).reshape(HIDDEN, W_CHARS * CD_PAD).T.astype(jnp.bfloat16)

    out = pl.pallas_call(
        _mm_body,
        grid=(N_TOK // _TB,),
        in_specs=[
            pl.BlockSpec((_TB, W_CHARS * CD_PAD), lambda i: (i, 0)),
            pl.BlockSpec((W_CHARS * CD_PAD, HIDDEN), lambda i: (0, 0)),
        ],
        out_specs=pl.BlockSpec((_TB, HIDDEN), lambda i: (i, 0)),
        out_shape=jax.ShapeDtypeStruct((N_TOK, HIDDEN), jnp.float32),
    )(emb.reshape(N_TOK, W_CHARS * CD_PAD), wt)
    return out.reshape(B, L, HIDDEN)
